# Initial kernel scaffold; baseline (speedup 1.0000x reference)
import jax
import jax.numpy as jnp
from jax.experimental import pallas as pl


def kernel(user_ids, item_ids, user_tables, item_tables, Wu1, bu1, Wu2, bu2, Wi1, bi1, Wi2, bi2, user_bias, item_bias):
    raise NotImplementedError("write your pallas kernel here")


# speedup vs baseline: 6.0920x; 6.0920x over previous
import functools

import jax
import jax.numpy as jnp
from jax import lax
from jax.experimental import pallas as pl
from jax.experimental.pallas import tpu as pltpu
from jax.experimental.pallas import tpu_sc as plsc

B = 4096
D = 768
K1 = 1024
K2 = 256
NC = 2
NS = 16
NW = NC * NS
BPW = B // NW
L = 16

BLK = 512


def _sc_gather_body(ut, it, ub, ib, uid0, uid1, iid0, iid1,
                    ug, ig, bsum,
                    idx_v, rows_v, bu_v, bi_v, sem):
    wid = lax.axis_index("s") * NC + lax.axis_index("c")
    base = wid * BPW
    sl = pl.ds(base, BPW)

    for ids_hbm, tab, out, c in ((uid0, ut, ug, 0), (uid1, ut, ug, 1),
                                 (iid0, it, ig, 0), (iid1, it, ig, 1)):
        pltpu.sync_copy(ids_hbm.at[sl], idx_v)
        pltpu.async_copy(tab.at[idx_v], rows_v, sem).wait()
        pltpu.sync_copy(rows_v, out.at[c, sl])

    pltpu.sync_copy(uid0.at[sl], idx_v)
    pltpu.async_copy(ub.at[idx_v], bu_v, sem).wait()
    pltpu.sync_copy(iid0.at[sl], idx_v)
    pltpu.async_copy(ib.at[idx_v], bi_v, sem).wait()
    for j in range(BPW // L):
        s = pl.ds(j * L, L)
        bu_v[s] = bu_v[s] + bi_v[s]
    pltpu.sync_copy(bu_v, bsum.at[sl])


def _make_sc_gather():
    return pl.kernel(
        _sc_gather_body,
        mesh=plsc.VectorSubcoreMesh(core_axis_name="c", subcore_axis_name="s"),
        out_type=[
            jax.ShapeDtypeStruct((2, B, D), jnp.float32),
            jax.ShapeDtypeStruct((2, B, D), jnp.float32),
            jax.ShapeDtypeStruct((B,), jnp.float32),
        ],
        scratch_types=[
            pltpu.VMEM((BPW,), jnp.int32),
            pltpu.VMEM((BPW, D), jnp.float32),
            pltpu.VMEM((BPW,), jnp.float32),
            pltpu.VMEM((BPW,), jnp.float32),
            pltpu.SemaphoreType.DMA,
        ],
    )


def _tc_ffn_body(ug_ref, ig_ref, wu1, bu1, wu2, bu2, wi1, bi1, wi2, bi2,
                 bsum_ref, out_ref):
    def two_layer(x, w1, b1, w2, b2):
        h = jnp.dot(x, w1[...], preferred_element_type=jnp.float32) + b1[...]
        h = jnp.maximum(h, 0.0)
        return jnp.dot(h, w2[...], preferred_element_type=jnp.float32) + b2[...]

    u = jnp.maximum(two_layer(ug_ref[0], wu1, bu1, wu2, bu2),
                    two_layer(ug_ref[1], wu1, bu1, wu2, bu2))
    v = jnp.maximum(two_layer(ig_ref[0], wi1, bi1, wi2, bi2),
                    two_layer(ig_ref[1], wi1, bi1, wi2, bi2))
    out_ref[...] = jnp.sum(u * v, axis=1) + bsum_ref[...]


_tc_ffn = pl.pallas_call(
    _tc_ffn_body,
    grid=(B // BLK,),
    in_specs=[
        pl.BlockSpec((2, BLK, D), lambda b: (0, b, 0)),
        pl.BlockSpec((2, BLK, D), lambda b: (0, b, 0)),
        pl.BlockSpec((D, K1), lambda b: (0, 0)),
        pl.BlockSpec((K1,), lambda b: (0,)),
        pl.BlockSpec((K1, K2), lambda b: (0, 0)),
        pl.BlockSpec((K2,), lambda b: (0,)),
        pl.BlockSpec((D, K1), lambda b: (0, 0)),
        pl.BlockSpec((K1,), lambda b: (0,)),
        pl.BlockSpec((K1, K2), lambda b: (0, 0)),
        pl.BlockSpec((K2,), lambda b: (0,)),
        pl.BlockSpec((BLK,), lambda b: (b,)),
    ],
    out_specs=pl.BlockSpec((BLK,), lambda b: (b,)),
    out_shape=jax.ShapeDtypeStruct((B,), jnp.float32),
)


def kernel(user_ids, item_ids, user_tables, item_tables, Wu1, bu1, Wu2, bu2,
           Wi1, bi1, Wi2, bi2, user_bias, item_bias):
    uids = user_ids[:, 0]
    iids = item_ids[:, 0]
    nu = user_tables.shape[1]
    ni = item_tables.shape[1]
    ut = user_tables.reshape(2 * nu, D)
    it = item_tables.reshape(2 * ni, D)
    ug, ig, bsum = _make_sc_gather()(ut, it, user_bias, item_bias,
                                     uids, uids + nu, iids, iids + ni)
    out = _tc_ffn(ug, ig, Wu1, bu1, Wu2, bu2, Wi1, bi1, Wi2, bi2, bsum)
    return out[:, None]
